# baseline (speedup 1.0000x reference)
import functools
import math

import jax
import jax.numpy as jnp
from jax import lax
from jax.experimental import pallas as pl
from jax.experimental.pallas import tpu as pltpu
from jax.experimental.pallas import tpu_sc as plsc

D_MODEL = 512
SCALE = math.sqrt(float(D_MODEL))
NG = 2
NS_RING = 3
UNROLL = 6


def _make_gather(V, D, B):
    info = plsc.get_sparse_core_info()
    NC, NS = info.num_cores, info.num_subcores
    NW = NC * NS
    assert B % NW == 0
    b_per_w = B // NW
    K = 40
    assert b_per_w % K == 0
    n_chunks = b_per_w // K
    tail = (n_chunks - UNROLL) % UNROLL
    n_groups = (n_chunks - UNROLL - tail) // UNROLL
    G = D // 16

    mesh = plsc.VectorSubcoreMesh(core_axis_name="c", subcore_axis_name="s")

    @functools.partial(
        pl.kernel,
        mesh=mesh,
        out_type=jax.ShapeDtypeStruct((B, D), jnp.float32),
        scratch_types=[
            pltpu.VMEM((n_chunks, K), jnp.int32),
            pltpu.VMEM((K, D), jnp.float32),
            pltpu.VMEM((K, D), jnp.float32),
            pltpu.VMEM((K, D), jnp.float32),
            pltpu.VMEM((K, D), jnp.float32),
            pltpu.VMEM((K, D), jnp.float32),
            pltpu.SemaphoreType.DMA,
            pltpu.SemaphoreType.DMA,
            pltpu.SemaphoreType.DMA,
            pltpu.SemaphoreType.DMA,
            pltpu.SemaphoreType.DMA,
        ],
    )
    def k(table, idx_hbm, out, idx_all, g0, g1, s0, s1, s2,
          gm0, gm1, sm0, sm1, sm2):
        wid = lax.axis_index("s") * NC + lax.axis_index("c")
        base = wid * b_per_w
        gbuf = [g0, g1]
        sbuf = [s0, s1, s2]
        gsem = [gm0, gm1]
        ssem = [sm0, sm1, sm2]

        pltpu.sync_copy(idx_hbm.at[wid], idx_all)

        def start_gather(gb, c):
            pltpu.async_copy(table.at[idx_all.at[c]], gbuf[gb], gsem[gb])

        def wait_gather(gb):
            pltpu.make_async_copy(table.at[idx_all.at[0]], gbuf[gb],
                                  gsem[gb]).wait()

        def start_store(sb, c):
            pltpu.async_copy(sbuf[sb], out.at[pl.ds(base + c * K, K)],
                             ssem[sb])

        def wait_store(sb):
            pltpu.make_async_copy(sbuf[sb], out.at[pl.ds(base, K)],
                                  ssem[sb]).wait()

        def scale(gb, sb):
            def row(r, carry):
                for j in range(G):
                    sl = pl.ds(j * 16, 16)
                    sbuf[sb][r, sl] = gbuf[gb][r, sl] * SCALE
                return carry

            lax.fori_loop(0, K, row, 0)

        def step(c, gb, sb, wait_st, prefetch):
            wait_gather(gb)
            if wait_st:
                wait_store(sb)
            scale(gb, sb)
            start_store(sb, c)
            if prefetch:
                start_gather(gb, c + NG)

        for c in range(NG):
            start_gather(c % NG, c)

        for c in range(UNROLL):
            step(c, c % NG, c % NS_RING, wait_st=(c >= NS_RING),
                 prefetch=True)

        def group(g, carry):
            c0 = UNROLL + UNROLL * g
            for j in range(UNROLL):
                step(c0 + j, j % NG, j % NS_RING, wait_st=True, prefetch=True)
            return carry

        lax.fori_loop(0, n_groups, group, 0)

        for c in range(n_chunks - tail, n_chunks):
            step(c, c % NG, c % NS_RING, wait_st=True,
                 prefetch=(c + NG < n_chunks))

        for sb in range(NS_RING):
            wait_store(sb)

    return k


def kernel(x, lut):
    Bdim, T = x.shape
    V, D = lut.shape
    B = Bdim * T
    info = plsc.get_sparse_core_info()
    NW = info.num_cores * info.num_subcores
    K = 40
    xf = x.reshape(NW, (B // NW) // K, K).astype(jnp.int32)
    out = _make_gather(V, D, B)(lut, xf)
    return out.reshape(Bdim, T, D)


# speedup vs baseline: 1.0099x; 1.0099x over previous
import functools
import math

import jax
import jax.numpy as jnp
from jax import lax
from jax.experimental import pallas as pl
from jax.experimental.pallas import tpu as pltpu
from jax.experimental.pallas import tpu_sc as plsc

D_MODEL = 512
SCALE = math.sqrt(float(D_MODEL))


def _make_gather(V, D, B):
    info = plsc.get_sparse_core_info()
    NC, NS = info.num_cores, info.num_subcores
    NW = NC * NS
    assert B % NW == 0
    b_per_w = B // NW
    K = 40
    assert b_per_w % (2 * K) == 0
    n_chunks = b_per_w // K
    n_pairs = n_chunks // 2
    G = D // 16

    mesh = plsc.VectorSubcoreMesh(core_axis_name="c", subcore_axis_name="s")

    @functools.partial(
        pl.kernel,
        mesh=mesh,
        out_type=jax.ShapeDtypeStruct((B, D), jnp.float32),
        scratch_types=[
            pltpu.VMEM((n_chunks, K), jnp.int32),
            pltpu.VMEM((K, D), jnp.float32),
            pltpu.VMEM((K, D), jnp.float32),
            pltpu.VMEM((K, D), jnp.float32),
            pltpu.VMEM((K, D), jnp.float32),
            pltpu.SemaphoreType.DMA,
            pltpu.SemaphoreType.DMA,
            pltpu.SemaphoreType.DMA,
            pltpu.SemaphoreType.DMA,
        ],
    )
    def k(table, idx_hbm, out, idx_all, g0, g1, s0, s1, gm0, gm1, sm0, sm1):
        wid = lax.axis_index("s") * NC + lax.axis_index("c")
        base = wid * b_per_w
        gbuf = [g0, g1]
        sbuf = [s0, s1]
        gsem = [gm0, gm1]
        ssem = [sm0, sm1]

        pltpu.sync_copy(idx_hbm.at[wid], idx_all)

        def start_gather(b, c):
            pltpu.async_copy(table.at[idx_all.at[c]], gbuf[b], gsem[b])

        def wait_gather(b):
            pltpu.make_async_copy(table.at[idx_all.at[0]], gbuf[b],
                                  gsem[b]).wait()

        def start_store(b, c):
            pltpu.async_copy(sbuf[b], out.at[pl.ds(base + c * K, K)], ssem[b])

        def wait_store(b):
            pltpu.make_async_copy(sbuf[b], out.at[pl.ds(base, K)],
                                  ssem[b]).wait()

        def scale(b):
            def row(r, carry):
                for j in range(G):
                    sl = pl.ds(j * 16, 16)
                    sbuf[b][r, sl] = gbuf[b][r, sl] * SCALE
                return carry

            lax.fori_loop(0, K, row, 0)

        for b in range(2):
            start_gather(b, b)

        for b in range(2):
            wait_gather(b)
            scale(b)
            start_store(b, b)
            start_gather(b, b + 2)

        def pair(p, carry):
            for b in range(2):
                c = 2 * p + b
                wait_gather(b)
                wait_store(b)
                scale(b)
                start_store(b, c)

                @pl.when(c + 2 < n_chunks)
                def _():
                    start_gather(b, c + 2)

            return carry

        lax.fori_loop(1, n_pairs, pair, 0)

        for b in range(2):
            wait_store(b)

    return k


def kernel(x, lut):
    Bdim, T = x.shape
    V, D = lut.shape
    B = Bdim * T
    info = plsc.get_sparse_core_info()
    NW = info.num_cores * info.num_subcores
    K = 40
    xf = x.reshape(NW, (B // NW) // K, K).astype(jnp.int32)
    out = _make_gather(V, D, B)(lut, xf)
    return out.reshape(Bdim, T, D)
